# Initial kernel scaffold; baseline (speedup 1.0000x reference)
import functools

import jax
import jax.numpy as jnp
from jax.experimental import pallas as pl
from jax.experimental.pallas import tpu as pltpu

_SCORE_THR = 0.001
_IOU_THR = 0.7
_MAX_DET = 300
_K = 2048
_BLK = 128
_OUT_PAD = 304


def _nms_kernel(ox1, oy1, ox2, oy2,
                ox1t, oy1t, ox2t, oy2t,
                s_row, s_col,
                f_row, f_col,
                lab_col, boxes_col,
                ob_ref, osc_ref, olab_ref, onv_ref,
                sup_ref, hi_ref):
    srow = s_row[...]
    frow = f_row[...]
    x1r = ox1[...]
    y1r = oy1[...]
    x2r = ox2[...]
    y2r = oy2[...]
    area_row = (x2r - x1r) * (y2r - y1r)

    def block(bi, _):
        sl = pl.ds(bi * _BLK, _BLK)
        cx1 = ox1t[sl, :]
        cy1 = oy1t[sl, :]
        cx2 = ox2t[sl, :]
        cy2 = oy2t[sl, :]
        x1m = jnp.maximum(cx1, x1r)
        y1m = jnp.maximum(cy1, y1r)
        x2m = jnp.minimum(cx2, x2r)
        y2m = jnp.minimum(cy2, y2r)
        inter = jnp.maximum(x2m - x1m, 0.0) * jnp.maximum(y2m - y1m, 0.0)
        area_col = (cx2 - cx1) * (cy2 - cy1)
        iou = inter / (area_col + area_row - inter + 1e-9)
        sup = iou > _IOU_THR
        sc = s_col[sl, :]
        fc = f_col[sl, :]
        higher = (sc > srow) | ((sc == srow) & (fc < frow))
        sup_ref[sl, :] = (sup & higher).astype(jnp.bfloat16)
        hi_ref[sl, :] = higher.astype(jnp.bfloat16)
        return 0

    jax.lax.fori_loop(0, _K // _BLK, block, 0, unroll=False)

    valid = srow > _SCORE_THR
    keep0 = valid.astype(jnp.float32)

    def cond(c):
        _, changed, it = c
        return changed & (it < _K + 2)

    def body(c):
        keep, _, it = c
        cnt = jnp.dot(keep.astype(jnp.bfloat16), sup_ref[...],
                      preferred_element_type=jnp.float32)
        new = jnp.where(valid & (cnt < 0.5), 1.0, 0.0)
        changed = jnp.any(new != keep)
        return new, changed, it + 1

    keep, _, _ = jax.lax.while_loop(cond, body, (keep0, True, 0))

    nkept = jnp.sum(keep)
    pos = jnp.dot(keep.astype(jnp.bfloat16), hi_ref[...],
                  preferred_element_type=jnp.float32)
    rows = jax.lax.broadcasted_iota(jnp.float32, (_OUT_PAD, _K), 0)
    onehot = jnp.where((rows == pos) & (keep > 0.5), 1.0, 0.0)

    ob_ref[...] = jnp.dot(onehot, boxes_col[...],
                          preferred_element_type=jnp.float32)
    osc_ref[...] = jnp.dot(onehot, s_col[...].astype(jnp.float32),
                           preferred_element_type=jnp.float32)
    lab_out = jnp.dot(onehot, lab_col[...], preferred_element_type=jnp.float32)
    slot = jax.lax.broadcasted_iota(jnp.float32, (_OUT_PAD, 1), 0)
    olab_ref[...] = jnp.where(slot < nkept, lab_out, -1.0).astype(jnp.int32)
    onv_ref[0, 0] = jnp.minimum(nkept, float(_MAX_DET)).astype(jnp.int32)


def kernel(boxes, scores):
    B, N, C = scores.shape
    flat = scores.reshape(N * C)
    masked = jnp.where(flat > _SCORE_THR, flat, -1.0)
    top_scores, top_idx = jax.lax.top_k(masked, _K)
    lab = top_idx % C
    box_idx = top_idx // C
    cb = boxes[0][box_idx]
    off = lab.astype(jnp.float32) * 1e4
    ocb = cb + off[:, None]

    row = lambda v: v.reshape(1, _K)
    col = lambda v: v.reshape(_K, 1)
    args = (
        row(ocb[:, 0]), row(ocb[:, 1]), row(ocb[:, 2]), row(ocb[:, 3]),
        col(ocb[:, 0]), col(ocb[:, 1]), col(ocb[:, 2]), col(ocb[:, 3]),
        row(top_scores), col(top_scores),
        row(top_idx), col(top_idx),
        col(lab.astype(jnp.float32)), cb,
    )

    ob, osc, olab, nv = pl.pallas_call(
        _nms_kernel,
        out_shape=[
            jax.ShapeDtypeStruct((_OUT_PAD, 4), jnp.float32),
            jax.ShapeDtypeStruct((_OUT_PAD, 1), jnp.float32),
            jax.ShapeDtypeStruct((_OUT_PAD, 1), jnp.int32),
            jax.ShapeDtypeStruct((1, 1), jnp.int32),
        ],
        scratch_shapes=[
            pltpu.VMEM((_K, _K), jnp.bfloat16),
            pltpu.VMEM((_K, _K), jnp.bfloat16),
        ],
    )(*args)

    out_boxes = ob[:_MAX_DET][None]
    out_scores = osc[:_MAX_DET, 0][None]
    out_labels = olab[:_MAX_DET, 0][None]
    n_valid = nv.reshape(1)
    return out_boxes, out_scores, out_labels, n_valid


# speedup vs baseline: 3.0499x; 3.0499x over previous
import functools

import jax
import jax.numpy as jnp
from jax.experimental import pallas as pl
from jax.experimental.pallas import tpu as pltpu

_SCORE_THR = 0.001
_IOU_THR = 0.7
_MAX_DET = 300
_K = 2048
_BLK = 128
_OUT_PAD = 304


def _nms_kernel(ox1, oy1, ox2, oy2,
                ox1t, oy1t, ox2t, oy2t,
                s_row, s_col,
                f_row, f_col,
                lab_col, boxes_col,
                ob_ref, osc_ref, olab_ref, onv_ref,
                sup_ref, hi_ref):
    srow = s_row[...]
    frow = f_row[...]
    x1r = ox1[...]
    y1r = oy1[...]
    x2r = ox2[...]
    y2r = oy2[...]
    area_row = (x2r - x1r) * (y2r - y1r)

    def block(bi, _):
        sl = pl.ds(bi * _BLK, _BLK)
        cx1 = ox1t[sl, :]
        cy1 = oy1t[sl, :]
        cx2 = ox2t[sl, :]
        cy2 = oy2t[sl, :]
        x1m = jnp.maximum(cx1, x1r)
        y1m = jnp.maximum(cy1, y1r)
        x2m = jnp.minimum(cx2, x2r)
        y2m = jnp.minimum(cy2, y2r)
        inter = jnp.maximum(x2m - x1m, 0.0) * jnp.maximum(y2m - y1m, 0.0)
        area_col = (cx2 - cx1) * (cy2 - cy1)
        iou = inter / (area_col + area_row - inter + 1e-9)
        sup = iou > _IOU_THR
        sc = s_col[sl, :]
        fc = f_col[sl, :]
        higher = (sc > srow) | ((sc == srow) & (fc < frow))
        sup_ref[sl, :] = (sup & higher).astype(jnp.bfloat16)
        hi_ref[sl, :] = higher.astype(jnp.bfloat16)
        return 0

    jax.lax.fori_loop(0, _K // _BLK, block, 0, unroll=False)

    valid = srow > _SCORE_THR
    keep0 = valid.astype(jnp.float32)

    def cond(c):
        _, changed, it = c
        return changed & (it < _K + 2)

    def body(c):
        keep, _, it = c
        cnt = jnp.dot(keep.astype(jnp.bfloat16), sup_ref[...],
                      preferred_element_type=jnp.float32)
        new = jnp.where(valid & (cnt < 0.5), 1.0, 0.0)
        changed = jnp.any(new != keep)
        return new, changed, it + 1

    keep, _, _ = jax.lax.while_loop(cond, body, (keep0, True, 0))

    nkept = jnp.sum(keep)
    pos = jnp.dot(keep.astype(jnp.bfloat16), hi_ref[...],
                  preferred_element_type=jnp.float32)
    rows = jax.lax.broadcasted_iota(jnp.int32, (_OUT_PAD, _K), 0)
    pos_i = pos.astype(jnp.int32)
    onehot = jnp.where((rows == pos_i) & (keep > 0.5), 1.0, 0.0)

    ob_ref[...] = jnp.dot(onehot, boxes_col[...],
                          preferred_element_type=jnp.float32)
    osc_ref[...] = jnp.dot(onehot, s_col[...].astype(jnp.float32),
                           preferred_element_type=jnp.float32)
    lab_out = jnp.dot(onehot, lab_col[...], preferred_element_type=jnp.float32)
    slot = jax.lax.broadcasted_iota(jnp.int32, (_OUT_PAD, 1), 0)
    olab_ref[...] = jnp.where(slot < nkept.astype(jnp.int32),
                              lab_out, -1.0).astype(jnp.int32)
    nv = jnp.minimum(nkept, float(_MAX_DET)).astype(jnp.int32)
    onv_ref[...] = jnp.zeros((1, 1), jnp.int32) + nv


def kernel(boxes, scores):
    B, N, C = scores.shape
    flat = scores.reshape(N * C)
    masked = jnp.where(flat > _SCORE_THR, flat, -1.0)
    top_scores, top_idx = jax.lax.top_k(masked, _K)
    lab = top_idx % C
    box_idx = top_idx // C
    cb = boxes[0][box_idx]
    off = lab.astype(jnp.float32) * 1e4
    ocb = cb + off[:, None]

    row = lambda v: v.reshape(1, _K)
    col = lambda v: v.reshape(_K, 1)
    args = (
        row(ocb[:, 0]), row(ocb[:, 1]), row(ocb[:, 2]), row(ocb[:, 3]),
        col(ocb[:, 0]), col(ocb[:, 1]), col(ocb[:, 2]), col(ocb[:, 3]),
        row(top_scores), col(top_scores),
        row(top_idx), col(top_idx),
        col(lab.astype(jnp.float32)), cb,
    )

    ob, osc, olab, nv = pl.pallas_call(
        _nms_kernel,
        out_shape=[
            jax.ShapeDtypeStruct((_OUT_PAD, 4), jnp.float32),
            jax.ShapeDtypeStruct((_OUT_PAD, 1), jnp.float32),
            jax.ShapeDtypeStruct((_OUT_PAD, 1), jnp.int32),
            jax.ShapeDtypeStruct((1, 1), jnp.int32),
        ],
        scratch_shapes=[
            pltpu.VMEM((_K, _K), jnp.bfloat16),
            pltpu.VMEM((_K, _K), jnp.bfloat16),
        ],
    )(*args)

    out_boxes = ob[:_MAX_DET][None]
    out_scores = osc[:_MAX_DET, 0][None]
    out_labels = olab[:_MAX_DET, 0][None]
    n_valid = nv.reshape(1)
    return out_boxes, out_scores, out_labels, n_valid


# speedup vs baseline: 31.9867x; 10.4877x over previous
import functools

import numpy as np
import jax
import jax.numpy as jnp
from jax import lax
from jax.experimental import pallas as pl
from jax.experimental.pallas import tpu as pltpu
from jax.experimental.pallas import tpu_sc as plsc

_SCORE_THR = 0.001
_IOU_THR = 0.7
_MAX_DET = 300
_K = 2048
_BLK = 128
_OUT_PAD = 304
_NS = 16
_HALF = 1280
_M = 2 * _HALF
_ROWS = 12544
_NPAD = _ROWS * 128
_GRP = _ROWS // 128
_RPW = _ROWS // 32
_PER_W = _RPW * 128
_SLOTS_W = _HALF // _NS
_TRASH = 1 << 16
_SPOOL = _HALF + _TRASH


def _plan_kernel(s_ref, vs_ref, sidx_ref, u_ref):
    s = s_ref[...]
    u_ref[...] = jnp.where(s > _SCORE_THR,
                           jax.lax.bitcast_convert_type(s, jnp.int32), 0)

    def body(_, carry):
        lo, hi = carry
        mid = lo + (hi - lo + 1) // 2
        c = jnp.sum((u_ref[...] >= mid).astype(jnp.int32))
        big = c >= _K
        return jnp.where(big, mid, lo), jnp.where(big, hi, mid - 1)

    lo, _ = jax.lax.fori_loop(0, 31, body,
                              (jnp.int32(0), jnp.int32(1 << 30)))
    vs_ref[...] = jnp.zeros((1, 1), jnp.int32) + lo

    mask = (u_ref[...] >= lo) & (u_ref[...] > 0)
    maskb = jnp.where(mask, 1.0, 0.0).astype(jnp.bfloat16)
    li = jax.lax.broadcasted_iota(jnp.int32, (128, 128), 0)
    lj = jax.lax.broadcasted_iota(jnp.int32, (128, 128), 1)
    tstrict = jnp.where(li < lj, 1.0, 0.0)
    rowpre = jnp.dot(maskb, tstrict.astype(jnp.bfloat16),
                     preferred_element_type=jnp.float32)
    rowpre3 = rowpre.reshape(_GRP, 128, 128)
    mask3 = mask.reshape(_GRP, 128, 128)
    cnt98 = jnp.sum(jnp.where(mask3, 1.0, 0.0), axis=2)
    gpre_in = jnp.dot(cnt98, tstrict,
                      preferred_element_type=jnp.float32)
    gsum = jnp.sum(cnt98, axis=1, keepdims=True)
    gi = jax.lax.broadcasted_iota(jnp.int32, (_GRP, _GRP), 0)
    gj = jax.lax.broadcasted_iota(jnp.int32, (_GRP, _GRP), 1)
    gt = jnp.where(gj < gi, 1.0, 0.0)
    gpre = jnp.dot(gt, gsum, preferred_element_type=jnp.float32)
    base98 = gpre_in + gpre

    gix = jax.lax.broadcasted_iota(jnp.int32, (_GRP, 128), 0)
    lix = jax.lax.broadcasted_iota(jnp.int32, (_GRP, 128), 1)
    b0 = jnp.sum(jnp.where((gix == _GRP // 2) & (lix == 0), base98, 0.0))
    incore1 = gix >= _GRP // 2
    basec = base98 - jnp.where(incore1, b0, 0.0)
    pos3 = rowpre3 + basec[:, :, None]
    posi = pos3.astype(jnp.int32)
    g3 = jax.lax.broadcasted_iota(jnp.int32, (_GRP, 128, 128), 0)
    i3 = jax.lax.broadcasted_iota(jnp.int32, (_GRP, 128, 128), 1)
    l3 = jax.lax.broadcasted_iota(jnp.int32, (_GRP, 128, 128), 2)
    fi = (g3 * 128 + i3) * 128 + l3
    trash = _HALF + jnp.bitwise_and(fi, _TRASH - 1)
    ok = mask3 & (posi < _HALF)
    sidx_ref[...] = jnp.where(ok, posi, trash)


def _plan(scores2d):
    return pl.pallas_call(
        _plan_kernel,
        out_shape=[
            jax.ShapeDtypeStruct((1, 1), jnp.int32),
            jax.ShapeDtypeStruct((_GRP, 128, 128), jnp.int32),
        ],
        scratch_shapes=[pltpu.VMEM((_ROWS, 128), jnp.int32)],
    )(scores2d)


def _sc_body(sidx_hbm, scores_hbm, bx1_hbm, by1_hbm, bx2_hbm, by2_hbm,
             flat_out, score_out, lab_out, ox1_out, oy1_out, ox2_out, oy2_out,
             sidx_v, arange_v, pool_sp, pf_v, gidx_v, bidx_v,
             sc128_v, g1_v, g2_v, g3_v, g4_v, outl_v, lab80_v, stage_v, sem):
    c = lax.axis_index("c")
    s = lax.axis_index("s")
    w = c * _NS + s
    row0 = w * _RPW
    e_base = w * _PER_W

    iota = lax.broadcasted_iota(jnp.int32, (_NS,), 0)
    zi = jnp.zeros((_NS,), jnp.int32)
    zf = jnp.zeros((_NS,), jnp.float32)

    stage_v[...] = zi - 1
    for j in range(_SLOTS_W // _NS):
        pltpu.sync_copy(stage_v, pool_sp.at[pl.ds(s * _SLOTS_W + j * _NS,
                                                  _NS)])
    plsc.subcore_barrier()

    pltpu.sync_copy(sidx_hbm.at[pl.ds(pl.multiple_of(row0, 8), _RPW)], sidx_v)

    def fill(j, _):
        arange_v[pl.ds(j * _NS, _NS)] = (zi + e_base) + j * _NS + iota
        return 0

    jax.lax.fori_loop(0, _PER_W // _NS, fill, 0)

    for b in range(0, _RPW, 56):
        cps = [pltpu.async_copy(arange_v.at[pl.ds(j * 128, 128)],
                                pool_sp.at[sidx_v.at[j]], sem)
               for j in range(b, min(b + 56, _RPW))]
        for cp in cps:
            cp.wait()
    plsc.subcore_barrier()

    sbase = s * _SLOTS_W
    gbase = c * _HALF + sbase
    pltpu.sync_copy(pool_sp.at[pl.ds(pl.multiple_of(sbase, _NS), _SLOTS_W)],
                    pf_v)
    for k in range(_SLOTS_W // _NS):
        fl = pf_v[pl.ds(k * _NS, _NS)]
        flc = jnp.minimum(jnp.maximum(fl, 0), 1599999)
        gidx_v[0, pl.ds(k * _NS, _NS)] = flc
        q = (flc.astype(jnp.float32) * jnp.float32(0.0125)).astype(jnp.int32)
        r = flc - q * 80
        q = jnp.where(r < 0, q - 1, q)
        r = flc - q * 80
        q = jnp.where(r >= 80, q + 1, q)
        bidx_v[0, pl.ds(k * _NS, _NS)] = jnp.minimum(jnp.maximum(q, 0), 19999)
    for k in range(_SLOTS_W // _NS, 128 // _NS):
        gidx_v[0, pl.ds(k * _NS, _NS)] = iota * 797 + k
        bidx_v[0, pl.ds(k * _NS, _NS)] = iota * 797 + k

    cps = [pltpu.async_copy(scores_hbm.at[gidx_v.at[0]], sc128_v, sem),
           pltpu.async_copy(bx1_hbm.at[bidx_v.at[0]], g1_v, sem),
           pltpu.async_copy(by1_hbm.at[bidx_v.at[0]], g2_v, sem),
           pltpu.async_copy(bx2_hbm.at[bidx_v.at[0]], g3_v, sem),
           pltpu.async_copy(by2_hbm.at[bidx_v.at[0]], g4_v, sem)]
    for cp in cps:
        cp.wait()

    for k in range(_SLOTS_W // _NS):
        sl = pl.ds(k * _NS, _NS)
        fl = pf_v[sl]
        empty = fl < 0
        outl_v[sl] = jnp.where(empty, zf - 1.0, sc128_v[sl])
        flc = jnp.minimum(jnp.maximum(fl, 0), 1599999)
        lab80_v[sl] = flc - bidx_v[0, sl] * 80
    off = pl.multiple_of(gbase, _NS)
    pltpu.sync_copy(pf_v, flat_out.at[pl.ds(off, _SLOTS_W)])
    pltpu.sync_copy(outl_v, score_out.at[pl.ds(off, _SLOTS_W)])
    pltpu.sync_copy(lab80_v, lab_out.at[pl.ds(off, _SLOTS_W)])
    pltpu.sync_copy(g1_v.at[pl.ds(0, _SLOTS_W)], ox1_out.at[pl.ds(off, _SLOTS_W)])
    pltpu.sync_copy(g2_v.at[pl.ds(0, _SLOTS_W)], oy1_out.at[pl.ds(off, _SLOTS_W)])
    pltpu.sync_copy(g3_v.at[pl.ds(0, _SLOTS_W)], ox2_out.at[pl.ds(off, _SLOTS_W)])
    pltpu.sync_copy(g4_v.at[pl.ds(0, _SLOTS_W)], oy2_out.at[pl.ds(off, _SLOTS_W)])


def _sc_select(sidx2d, scores_flat, bx1, by1, bx2, by2):
    mesh = plsc.VectorSubcoreMesh(core_axis_name="c", subcore_axis_name="s")
    f = functools.partial(
        pl.kernel,
        mesh=mesh,
        out_type=[
            jax.ShapeDtypeStruct((_M,), jnp.int32),
            jax.ShapeDtypeStruct((_M,), jnp.float32),
            jax.ShapeDtypeStruct((_M,), jnp.int32),
            jax.ShapeDtypeStruct((_M,), jnp.float32),
            jax.ShapeDtypeStruct((_M,), jnp.float32),
            jax.ShapeDtypeStruct((_M,), jnp.float32),
            jax.ShapeDtypeStruct((_M,), jnp.float32),
        ],
        scratch_types=[
            pltpu.VMEM((_RPW, 128), jnp.int32),
            pltpu.VMEM((_PER_W,), jnp.int32),
            pltpu.VMEM_SHARED((_SPOOL,), jnp.int32),
            pltpu.VMEM((_SLOTS_W,), jnp.int32),
            pltpu.VMEM((1, 128), jnp.int32),
            pltpu.VMEM((1, 128), jnp.int32),
            pltpu.VMEM((128,), jnp.float32),
            pltpu.VMEM((128,), jnp.float32),
            pltpu.VMEM((128,), jnp.float32),
            pltpu.VMEM((128,), jnp.float32),
            pltpu.VMEM((128,), jnp.float32),
            pltpu.VMEM((_SLOTS_W,), jnp.float32),
            pltpu.VMEM((_SLOTS_W,), jnp.int32),
            pltpu.VMEM((_NS,), jnp.int32),
            pltpu.SemaphoreType.DMA,
        ],
    )(_sc_body)
    return f(sidx2d, scores_flat, bx1, by1, bx2, by2)


def _nms_kernel(ox1, oy1, ox2, oy2,
                ox1t, oy1t, ox2t, oy2t,
                s_row, s_col,
                f_row, f_col,
                lab_col, boxes_col,
                ob_ref, osc_ref, olab_ref, onv_ref,
                sup_ref, hi_ref):
    srow = s_row[...]
    frow = f_row[...]
    x1r = ox1[...]
    y1r = oy1[...]
    x2r = ox2[...]
    y2r = oy2[...]
    area_row = (x2r - x1r) * (y2r - y1r)

    def block(bi, _):
        sl = pl.ds(bi * _BLK, _BLK)
        cx1 = ox1t[sl, :]
        cy1 = oy1t[sl, :]
        cx2 = ox2t[sl, :]
        cy2 = oy2t[sl, :]
        x1m = jnp.maximum(cx1, x1r)
        y1m = jnp.maximum(cy1, y1r)
        x2m = jnp.minimum(cx2, x2r)
        y2m = jnp.minimum(cy2, y2r)
        inter = jnp.maximum(x2m - x1m, 0.0) * jnp.maximum(y2m - y1m, 0.0)
        area_col = (cx2 - cx1) * (cy2 - cy1)
        iou = inter / (area_col + area_row - inter + 1e-9)
        sup = iou > _IOU_THR
        sc = s_col[sl, :]
        fc = f_col[sl, :]
        higher = (sc > srow) | ((sc == srow) & (fc < frow))
        sup_ref[sl, :] = (sup & higher).astype(jnp.bfloat16)
        hi_ref[sl, :] = higher.astype(jnp.bfloat16)
        return 0

    jax.lax.fori_loop(0, _M // _BLK, block, 0, unroll=False)

    ones = jnp.zeros((1, _M), jnp.bfloat16) + jnp.bfloat16(1)
    rank = jnp.dot(ones, hi_ref[...], preferred_element_type=jnp.float32)
    valid = (srow > _SCORE_THR) & (rank < float(_K))
    keep0 = valid.astype(jnp.float32)

    def cond(c):
        _, changed, it = c
        return changed & (it < _M + 2)

    def body(c):
        keep, _, it = c
        cnt = jnp.dot(keep.astype(jnp.bfloat16), sup_ref[...],
                      preferred_element_type=jnp.float32)
        new = jnp.where(valid & (cnt < 0.5), 1.0, 0.0)
        changed = jnp.any(new != keep)
        return new, changed, it + 1

    keep, _, _ = jax.lax.while_loop(cond, body, (keep0, True, 0))

    nkept = jnp.sum(keep)
    pos = jnp.dot(keep.astype(jnp.bfloat16), hi_ref[...],
                  preferred_element_type=jnp.float32)
    rows = jax.lax.broadcasted_iota(jnp.int32, (_OUT_PAD, _M), 0)
    pos_i = pos.astype(jnp.int32)
    onehot = jnp.where((rows == pos_i) & (keep > 0.5), 1.0, 0.0)

    ob_ref[...] = jnp.dot(onehot, boxes_col[...],
                          preferred_element_type=jnp.float32)
    osc_ref[...] = jnp.dot(onehot, s_col[...].astype(jnp.float32),
                           preferred_element_type=jnp.float32)
    lab_out = jnp.dot(onehot, lab_col[...], preferred_element_type=jnp.float32)
    slot = jax.lax.broadcasted_iota(jnp.int32, (_OUT_PAD, 1), 0)
    olab_ref[...] = jnp.where(slot < nkept.astype(jnp.int32),
                              lab_out, -1.0).astype(jnp.int32)
    nv = jnp.minimum(nkept, float(_MAX_DET)).astype(jnp.int32)
    onv_ref[...] = jnp.zeros((1, 1), jnp.int32) + nv


def _nms(cand_boxes, cand_scores, cand_flat, cand_labels):
    off = cand_labels.astype(jnp.float32) * 1e4
    ocb = cand_boxes + off[:, None]

    row = lambda v: v.reshape(1, _M)
    col = lambda v: v.reshape(_M, 1)
    args = (
        row(ocb[:, 0]), row(ocb[:, 1]), row(ocb[:, 2]), row(ocb[:, 3]),
        col(ocb[:, 0]), col(ocb[:, 1]), col(ocb[:, 2]), col(ocb[:, 3]),
        row(cand_scores), col(cand_scores),
        row(cand_flat), col(cand_flat),
        col(cand_labels.astype(jnp.float32)), cand_boxes,
    )
    return pl.pallas_call(
        _nms_kernel,
        out_shape=[
            jax.ShapeDtypeStruct((_OUT_PAD, 4), jnp.float32),
            jax.ShapeDtypeStruct((_OUT_PAD, 1), jnp.float32),
            jax.ShapeDtypeStruct((_OUT_PAD, 1), jnp.int32),
            jax.ShapeDtypeStruct((1, 1), jnp.int32),
        ],
        scratch_shapes=[
            pltpu.VMEM((_M, _M), jnp.bfloat16),
            pltpu.VMEM((_M, _M), jnp.bfloat16),
        ],
    )(*args)


def kernel(boxes, scores):
    B, N, C = scores.shape
    flat_scores = scores.reshape(N * C)
    spad = jnp.pad(flat_scores, (0, _NPAD - N * C))
    vs, sidx3 = _plan(spad.reshape(_ROWS, 128))
    sidx = sidx3.reshape(_ROWS, 128)
    cflat, cscore, clab, cx1, cy1, cx2, cy2 = _sc_select(
        sidx, flat_scores,
        boxes[0, :, 0], boxes[0, :, 1], boxes[0, :, 2], boxes[0, :, 3])
    cboxes = jnp.stack([cx1, cy1, cx2, cy2], axis=1)
    ob, osc, olab, nv = _nms(cboxes, cscore, cflat, clab)

    out_boxes = ob[:_MAX_DET][None]
    out_scores = osc[:_MAX_DET, 0][None]
    out_labels = olab[:_MAX_DET, 0][None]
    n_valid = nv.reshape(1)
    return out_boxes, out_scores, out_labels, n_valid


# speedup vs baseline: 32.7292x; 1.0232x over previous
import functools

import numpy as np
import jax
import jax.numpy as jnp
from jax import lax
from jax.experimental import pallas as pl
from jax.experimental.pallas import tpu as pltpu
from jax.experimental.pallas import tpu_sc as plsc

_SCORE_THR = 0.001
_IOU_THR = 0.7
_MAX_DET = 300
_K = 2048
_BLK = 128
_OUT_PAD = 304
_NS = 16
_HALF = 1280
_M = 2 * _HALF
_ROWS = 12544
_NPAD = _ROWS * 128
_GRP = _ROWS // 128
_RPW = _ROWS // 32
_PER_W = _RPW * 128
_SLOTS_W = _HALF // _NS
_TRASH = 1 << 16
_SPOOL = _HALF + _TRASH


def _plan_kernel(s_ref, vs_ref, sidx_ref, u_ref):
    s = s_ref[...]
    u_ref[...] = jnp.where(s > _SCORE_THR,
                           jax.lax.bitcast_convert_type(s, jnp.int32), 0)
    ones128 = jnp.zeros((128, 1), jnp.bfloat16) + jnp.bfloat16(1)

    def count_ge(v):
        cb = jnp.where(u_ref[...] >= v, 1.0, 0.0).astype(jnp.bfloat16)
        c1 = jnp.dot(cb, ones128, preferred_element_type=jnp.float32)
        return jnp.sum(c1)

    u_thr = np.float32(_SCORE_THR).view(np.int32).item()
    many = count_ge(jnp.int32(u_thr + 1)) >= float(_K)

    def body(_, carry):
        lo, hi = carry
        mid = lo + (hi - lo + 1) // 2
        big = count_ge(mid) >= float(_K)
        return jnp.where(big, mid, lo), jnp.where(big, hi, mid - 1)

    lo0 = jnp.where(many, jnp.int32(u_thr + 1), jnp.int32(0))
    hi0 = jnp.where(many, jnp.int32(0x3F7FFFFF), jnp.int32(u_thr))
    lo, _ = jax.lax.fori_loop(0, 27, body, (lo0, hi0))
    lo = jnp.where(many, lo, 0)
    vs_ref[...] = jnp.zeros((1, 1), jnp.int32) + lo

    mask = (u_ref[...] >= lo) & (u_ref[...] > 0)
    maskb = jnp.where(mask, 1.0, 0.0).astype(jnp.bfloat16)
    li = jax.lax.broadcasted_iota(jnp.int32, (128, 128), 0)
    lj = jax.lax.broadcasted_iota(jnp.int32, (128, 128), 1)
    tstrict = jnp.where(li < lj, 1.0, 0.0)
    rowpre = jnp.dot(maskb, tstrict.astype(jnp.bfloat16),
                     preferred_element_type=jnp.float32)
    rowpre3 = rowpre.reshape(32, _RPW, 128)
    mask3 = mask.reshape(32, _RPW, 128)
    cntw = jnp.sum(jnp.where(mask3, 1.0, 0.0), axis=2)
    ri = jax.lax.broadcasted_iota(jnp.int32, (_RPW, _RPW), 0)
    rj = jax.lax.broadcasted_iota(jnp.int32, (_RPW, _RPW), 1)
    t392 = jnp.where(ri < rj, 1.0, 0.0)
    gpre_in = jnp.dot(cntw, t392,
                      preferred_element_type=jnp.float32)
    gsum = jnp.sum(cntw, axis=1, keepdims=True)
    gi = jax.lax.broadcasted_iota(jnp.int32, (32, 32), 0)
    gj = jax.lax.broadcasted_iota(jnp.int32, (32, 32), 1)
    gt = jnp.where(gj < gi, 1.0, 0.0)
    gpre = jnp.dot(gt, gsum, preferred_element_type=jnp.float32)
    base = gpre_in + gpre

    gix = jax.lax.broadcasted_iota(jnp.int32, (32, _RPW), 0)
    rix = jax.lax.broadcasted_iota(jnp.int32, (32, _RPW), 1)
    b0 = jnp.sum(jnp.where((gix == 16) & (rix == 0), base, 0.0))
    basec = base - jnp.where(gix >= 16, b0, 0.0)
    pos3 = rowpre3 + basec[:, :, None]
    posi = pos3.astype(jnp.int32)
    g3 = jax.lax.broadcasted_iota(jnp.int32, (32, _RPW, 128), 0)
    i3 = jax.lax.broadcasted_iota(jnp.int32, (32, _RPW, 128), 1)
    l3 = jax.lax.broadcasted_iota(jnp.int32, (32, _RPW, 128), 2)
    fi = (g3 * _RPW + i3) * 128 + l3
    trash = _HALF + jnp.bitwise_and(fi, _TRASH - 1)
    ok = mask3 & (posi < _HALF)
    sidx_ref[...] = jnp.where(ok, posi, trash)


def _plan(scores2d):
    return pl.pallas_call(
        _plan_kernel,
        out_shape=[
            jax.ShapeDtypeStruct((1, 1), jnp.int32),
            jax.ShapeDtypeStruct((32, _RPW, 128), jnp.int32),
        ],
        scratch_shapes=[pltpu.VMEM((_ROWS, 128), jnp.int32)],
    )(scores2d)


def _sc_body(sidx_hbm, arange_hbm, scores_hbm, bx1_hbm, by1_hbm, bx2_hbm,
             by2_hbm,
             flat_out, score_out, lab_out, ox1_out, oy1_out, ox2_out, oy2_out,
             sidx_v, arange_v, pool_sp, pf_v, gidx_v, bidx_v,
             sc128_v, g1_v, g2_v, g3_v, g4_v, outl_v, lab80_v, stage_v,
             sem, sem2):
    c = lax.axis_index("c")
    s = lax.axis_index("s")
    w = c * _NS + s
    e_base = w * _PER_W

    iota = lax.broadcasted_iota(jnp.int32, (_NS,), 0)
    zi = jnp.zeros((_NS,), jnp.int32)
    zf = jnp.zeros((_NS,), jnp.float32)

    cp_sidx = pltpu.async_copy(sidx_hbm.at[w], sidx_v, sem2)
    cp_ar = pltpu.async_copy(
        arange_hbm.at[pl.ds(pl.multiple_of(e_base, _NS), _PER_W)],
        arange_v, sem2)

    stage_v[...] = zi - 1
    for j in range(_SLOTS_W // _NS):
        pltpu.sync_copy(stage_v, pool_sp.at[pl.ds(s * _SLOTS_W + j * _NS,
                                                  _NS)])
    plsc.subcore_barrier()
    cp_sidx.wait()
    cp_ar.wait()

    for b in range(0, _RPW, 56):
        cps = [pltpu.async_copy(arange_v.at[pl.ds(j * 128, 128)],
                                pool_sp.at[sidx_v.at[j]], sem)
               for j in range(b, min(b + 56, _RPW))]
        for cp in cps:
            cp.wait()
    plsc.subcore_barrier()

    sbase = s * _SLOTS_W
    gbase = c * _HALF + sbase
    pltpu.sync_copy(pool_sp.at[pl.ds(pl.multiple_of(sbase, _NS), _SLOTS_W)],
                    pf_v)
    for k in range(_SLOTS_W // _NS):
        fl = pf_v[pl.ds(k * _NS, _NS)]
        flc = jnp.minimum(jnp.maximum(fl, 0), 1599999)
        gidx_v[0, pl.ds(k * _NS, _NS)] = flc
        q = (flc.astype(jnp.float32) * jnp.float32(0.0125)).astype(jnp.int32)
        r = flc - q * 80
        q = jnp.where(r < 0, q - 1, q)
        r = flc - q * 80
        q = jnp.where(r >= 80, q + 1, q)
        bidx_v[0, pl.ds(k * _NS, _NS)] = jnp.minimum(jnp.maximum(q, 0), 19999)
    for k in range(_SLOTS_W // _NS, 128 // _NS):
        gidx_v[0, pl.ds(k * _NS, _NS)] = iota * 797 + k
        bidx_v[0, pl.ds(k * _NS, _NS)] = iota * 797 + k

    cps = [pltpu.async_copy(scores_hbm.at[gidx_v.at[0]], sc128_v, sem),
           pltpu.async_copy(bx1_hbm.at[bidx_v.at[0]], g1_v, sem),
           pltpu.async_copy(by1_hbm.at[bidx_v.at[0]], g2_v, sem),
           pltpu.async_copy(bx2_hbm.at[bidx_v.at[0]], g3_v, sem),
           pltpu.async_copy(by2_hbm.at[bidx_v.at[0]], g4_v, sem)]
    for cp in cps:
        cp.wait()

    for k in range(_SLOTS_W // _NS):
        sl = pl.ds(k * _NS, _NS)
        fl = pf_v[sl]
        empty = fl < 0
        outl_v[sl] = jnp.where(empty, zf - 1.0, sc128_v[sl])
        flc = jnp.minimum(jnp.maximum(fl, 0), 1599999)
        lab80_v[sl] = flc - bidx_v[0, sl] * 80
    off = pl.multiple_of(gbase, _NS)
    pltpu.sync_copy(pf_v, flat_out.at[pl.ds(off, _SLOTS_W)])
    pltpu.sync_copy(outl_v, score_out.at[pl.ds(off, _SLOTS_W)])
    pltpu.sync_copy(lab80_v, lab_out.at[pl.ds(off, _SLOTS_W)])
    pltpu.sync_copy(g1_v.at[pl.ds(0, _SLOTS_W)], ox1_out.at[pl.ds(off, _SLOTS_W)])
    pltpu.sync_copy(g2_v.at[pl.ds(0, _SLOTS_W)], oy1_out.at[pl.ds(off, _SLOTS_W)])
    pltpu.sync_copy(g3_v.at[pl.ds(0, _SLOTS_W)], ox2_out.at[pl.ds(off, _SLOTS_W)])
    pltpu.sync_copy(g4_v.at[pl.ds(0, _SLOTS_W)], oy2_out.at[pl.ds(off, _SLOTS_W)])


def _sc_select(sidx3, arange, scores_flat, bx1, by1, bx2, by2):
    mesh = plsc.VectorSubcoreMesh(core_axis_name="c", subcore_axis_name="s")
    f = functools.partial(
        pl.kernel,
        mesh=mesh,
        out_type=[
            jax.ShapeDtypeStruct((_M,), jnp.int32),
            jax.ShapeDtypeStruct((_M,), jnp.float32),
            jax.ShapeDtypeStruct((_M,), jnp.int32),
            jax.ShapeDtypeStruct((_M,), jnp.float32),
            jax.ShapeDtypeStruct((_M,), jnp.float32),
            jax.ShapeDtypeStruct((_M,), jnp.float32),
            jax.ShapeDtypeStruct((_M,), jnp.float32),
        ],
        scratch_types=[
            pltpu.VMEM((_RPW, 128), jnp.int32),
            pltpu.VMEM((_PER_W,), jnp.int32),
            pltpu.VMEM_SHARED((_SPOOL,), jnp.int32),
            pltpu.VMEM((_SLOTS_W,), jnp.int32),
            pltpu.VMEM((1, 128), jnp.int32),
            pltpu.VMEM((1, 128), jnp.int32),
            pltpu.VMEM((128,), jnp.float32),
            pltpu.VMEM((128,), jnp.float32),
            pltpu.VMEM((128,), jnp.float32),
            pltpu.VMEM((128,), jnp.float32),
            pltpu.VMEM((128,), jnp.float32),
            pltpu.VMEM((_SLOTS_W,), jnp.float32),
            pltpu.VMEM((_SLOTS_W,), jnp.int32),
            pltpu.VMEM((_NS,), jnp.int32),
            pltpu.SemaphoreType.DMA,
            pltpu.SemaphoreType.DMA,
        ],
    )(_sc_body)
    return f(sidx3, arange, scores_flat, bx1, by1, bx2, by2)


def _nms_kernel(ox1, oy1, ox2, oy2,
                ox1t, oy1t, ox2t, oy2t,
                s_row, s_col,
                f_row, f_col,
                lab_col, boxes_col,
                ob_ref, osc_ref, olab_ref, onv_ref,
                sup_ref, hi_ref):
    srow = s_row[...]
    frow = f_row[...]
    x1r = ox1[...]
    y1r = oy1[...]
    x2r = ox2[...]
    y2r = oy2[...]
    area_row = (x2r - x1r) * (y2r - y1r)

    def block(bi, _):
        sl = pl.ds(bi * _BLK, _BLK)
        cx1 = ox1t[sl, :]
        cy1 = oy1t[sl, :]
        cx2 = ox2t[sl, :]
        cy2 = oy2t[sl, :]
        x1m = jnp.maximum(cx1, x1r)
        y1m = jnp.maximum(cy1, y1r)
        x2m = jnp.minimum(cx2, x2r)
        y2m = jnp.minimum(cy2, y2r)
        inter = jnp.maximum(x2m - x1m, 0.0) * jnp.maximum(y2m - y1m, 0.0)
        area_col = (cx2 - cx1) * (cy2 - cy1)
        iou = inter / (area_col + area_row - inter + 1e-9)
        sup = iou > _IOU_THR
        sc = s_col[sl, :]
        fc = f_col[sl, :]
        higher = (sc > srow) | ((sc == srow) & (fc < frow))
        sup_ref[sl, :] = (sup & higher).astype(jnp.bfloat16)
        hi_ref[sl, :] = higher.astype(jnp.bfloat16)
        return 0

    jax.lax.fori_loop(0, _M // _BLK, block, 0, unroll=False)

    ones = jnp.zeros((1, _M), jnp.bfloat16) + jnp.bfloat16(1)
    rank = jnp.dot(ones, hi_ref[...], preferred_element_type=jnp.float32)
    valid = (srow > _SCORE_THR) & (rank < float(_K))
    keep0 = valid.astype(jnp.float32)

    def cond(c):
        _, changed, it = c
        return changed & (it < _M + 2)

    def body(c):
        keep, _, it = c
        cnt = jnp.dot(keep.astype(jnp.bfloat16), sup_ref[...],
                      preferred_element_type=jnp.float32)
        new = jnp.where(valid & (cnt < 0.5), 1.0, 0.0)
        changed = jnp.any(new != keep)
        return new, changed, it + 1

    keep, _, _ = jax.lax.while_loop(cond, body, (keep0, True, 0))

    nkept = jnp.sum(keep)
    pos = jnp.dot(keep.astype(jnp.bfloat16), hi_ref[...],
                  preferred_element_type=jnp.float32)
    rows = jax.lax.broadcasted_iota(jnp.int32, (_OUT_PAD, _M), 0)
    pos_i = pos.astype(jnp.int32)
    onehot = jnp.where((rows == pos_i) & (keep > 0.5), 1.0, 0.0)

    ob_ref[...] = jnp.dot(onehot, boxes_col[...],
                          preferred_element_type=jnp.float32)
    osc_ref[...] = jnp.dot(onehot, s_col[...].astype(jnp.float32),
                           preferred_element_type=jnp.float32)
    lab_out = jnp.dot(onehot, lab_col[...], preferred_element_type=jnp.float32)
    slot = jax.lax.broadcasted_iota(jnp.int32, (_OUT_PAD, 1), 0)
    olab_ref[...] = jnp.where(slot < nkept.astype(jnp.int32),
                              lab_out, -1.0).astype(jnp.int32)
    nv = jnp.minimum(nkept, float(_MAX_DET)).astype(jnp.int32)
    onv_ref[...] = jnp.zeros((1, 1), jnp.int32) + nv


def _nms(cand_boxes, cand_scores, cand_flat, cand_labels):
    off = cand_labels.astype(jnp.float32) * 1e4
    ocb = cand_boxes + off[:, None]

    row = lambda v: v.reshape(1, _M)
    col = lambda v: v.reshape(_M, 1)
    args = (
        row(ocb[:, 0]), row(ocb[:, 1]), row(ocb[:, 2]), row(ocb[:, 3]),
        col(ocb[:, 0]), col(ocb[:, 1]), col(ocb[:, 2]), col(ocb[:, 3]),
        row(cand_scores), col(cand_scores),
        row(cand_flat), col(cand_flat),
        col(cand_labels.astype(jnp.float32)), cand_boxes,
    )
    return pl.pallas_call(
        _nms_kernel,
        out_shape=[
            jax.ShapeDtypeStruct((_OUT_PAD, 4), jnp.float32),
            jax.ShapeDtypeStruct((_OUT_PAD, 1), jnp.float32),
            jax.ShapeDtypeStruct((_OUT_PAD, 1), jnp.int32),
            jax.ShapeDtypeStruct((1, 1), jnp.int32),
        ],
        scratch_shapes=[
            pltpu.VMEM((_M, _M), jnp.bfloat16),
            pltpu.VMEM((_M, _M), jnp.bfloat16),
        ],
    )(*args)


def kernel(boxes, scores):
    B, N, C = scores.shape
    flat_scores = scores.reshape(N * C)
    spad = jnp.pad(flat_scores, (0, _NPAD - N * C))
    vs, sidx3 = _plan(spad.reshape(_ROWS, 128))
    arange = jnp.arange(_NPAD, dtype=jnp.int32)
    cflat, cscore, clab, cx1, cy1, cx2, cy2 = _sc_select(
        sidx3, arange, flat_scores,
        boxes[0, :, 0], boxes[0, :, 1], boxes[0, :, 2], boxes[0, :, 3])
    cboxes = jnp.stack([cx1, cy1, cx2, cy2], axis=1)
    ob, osc, olab, nv = _nms(cboxes, cscore, cflat, clab)

    out_boxes = ob[:_MAX_DET][None]
    out_scores = osc[:_MAX_DET, 0][None]
    out_labels = olab[:_MAX_DET, 0][None]
    n_valid = nv.reshape(1)
    return out_boxes, out_scores, out_labels, n_valid
